# baseline (speedup 1.0000x reference)
import functools

import jax
import jax.numpy as jnp
from jax import lax
from jax.experimental import pallas as pl
from jax.experimental.pallas import tpu as pltpu
from jax.experimental.pallas import tpu_sc as plsc

D = 128
W = 2 * D + 1
WP = 272
S_DIM, B_DIM, H_DIM = 2048, 32, 1024
CH = 16
NCH = WP // CH
NBUF = 4
LOOKAHEAD = 3
NQ = H_DIM // 16


def _predict_kernel(hid_ref, wp_ref, wpb_ref, vp_ref, vpb_ref,
                    idx_ref, gauss_ref, live_ref):
    h = hid_ref[...]
    wph = lax.dot_general(h, wp_ref[...], (((1,), (1,)), ((), ())),
                          preferred_element_type=jnp.float32)
    wph = jnp.tanh(wph + wpb_ref[...])
    vp8 = jnp.broadcast_to(vp_ref[...], (8, H_DIM))
    z = lax.dot_general(wph, vp8, (((1,), (1,)), ((), ())),
                        preferred_element_type=jnp.float32)[:, :1]
    p = S_DIM * jax.nn.sigmoid(z + vpb_ref[0, 0])
    c = lax.round(p, lax.RoundingMethod.TO_NEAREST_EVEN).astype(jnp.int32)
    j = lax.broadcasted_iota(jnp.int32, (B_DIM, WP), 1)
    b = lax.broadcasted_iota(jnp.int32, (B_DIM, WP), 0)
    s_clamped = jnp.clip(c - D + j, 0, S_DIM - 1)
    idx_ref[...] = s_clamped * B_DIM + b
    j2 = lax.broadcasted_iota(jnp.int32, (B_DIM, WP), 1)
    s_abs = c - D + j2
    live = (s_abs >= 0) & (s_abs < S_DIM) & (j2 < W)
    live_ref[...] = live.astype(jnp.float32)
    wi = s_abs.astype(jnp.float32)
    gauss_ref[...] = jnp.exp((wi - p) ** 2 * (-1.0 / 8192.0))


def _make_sc_attend():
    mesh = plsc.VectorSubcoreMesh(core_axis_name="c", subcore_axis_name="s")
    info = plsc.get_sparse_core_info()
    nc = info.num_cores

    @functools.partial(
        pl.kernel, mesh=mesh,
        out_type=[
            jax.ShapeDtypeStruct((B_DIM * WP,), jnp.float32),
            jax.ShapeDtypeStruct((B_DIM * H_DIM,), jnp.float32),
        ],
        scratch_types=(
            [pltpu.VMEM((WP,), jnp.int32),
             pltpu.VMEM((H_DIM,), jnp.float32),
             pltpu.VMEM((WP,), jnp.float32),
             pltpu.VMEM((WP,), jnp.float32),
             pltpu.VMEM((WP,), jnp.float32),
             pltpu.VMEM((WP,), jnp.float32),
             pltpu.VMEM((H_DIM,), jnp.float32)]
            + [pltpu.VMEM((CH, H_DIM), jnp.float32)] * NBUF
            + [pltpu.SemaphoreType.DMA] * NBUF
        ),
    )
    def attend_k(idx_hbm, hid_hbm, gauss_hbm, live_hbm, table_hbm,
                 scaled_hbm, ctx_hbm,
                 idx_v, hv, gauss_v, live_v, sc_v, so_v, ctx_v, *scr):
        bufs = list(scr[:NBUF])
        gsem = list(scr[NBUF:])
        zero16 = jnp.zeros((16,), jnp.float32)
        wid = lax.axis_index("s") * nc + lax.axis_index("c")
        pltpu.sync_copy(idx_hbm.at[pl.ds(wid * WP, WP)], idx_v)
        pltpu.sync_copy(hid_hbm.at[pl.ds(wid * H_DIM, H_DIM)], hv)
        pltpu.sync_copy(gauss_hbm.at[pl.ds(wid * WP, WP)], gauss_v)
        pltpu.sync_copy(live_hbm.at[pl.ds(wid * WP, WP)], live_v)
        sc_v[pl.ds(WP - 16, 16)] = zero16

        lane = lax.iota(jnp.int32, 16)

        def rnd(x):
            c = x * 65537.0
            return c - (c - x)

        def rnd2(a, b):
            return rnd(a), rnd(b)

        def hstep(q, carry):
            off = pl.ds(q * 16, 16)
            hv[off] = rnd(hv[off])
            return carry
        lax.fori_loop(0, NQ, hstep, 0)

        gd = lax.GatherDimensionNumbers(offset_dims=(), collapsed_slice_dims=(0,),
                                        start_index_map=(0,))

        def _perm(x, idx):
            return lax.gather(x, idx[:, None], gd, slice_sizes=(1,),
                              mode=lax.GatherScatterMode.PROMISE_IN_BOUNDS)

        def _bcast(x, r):
            return _perm(x, jnp.full((16,), r, jnp.int32))

        def _bfly(x, op):
            for sh in (8, 4, 2, 1):
                x = op(x, _perm(x, lane ^ sh))
            return x

        def zstep(q, carry):
            ctx_v[pl.ds(q * 16, 16)] = zero16
            return carry
        lax.fori_loop(0, NQ, zstep, 0)

        m_run = [zero16]

        def attend_chunk(buf, cj):
            off16 = pl.ds(cj * CH, CH)

            def qstep(q, accs):
                off = pl.ds(q * 16, 16)
                hq = hv[off]
                out = []
                for r in range(0, CH, 2):
                    ra, rb = rnd2(buf[r, off], buf[r + 1, off])
                    buf[r, off] = ra
                    buf[r + 1, off] = rb
                    out.append(accs[r] + ra * hq)
                    out.append(accs[r + 1] + rb * hq)
                return tuple(out)
            accs = lax.fori_loop(0, NQ, qstep, (zero16,) * CH)
            row_scores = zero16
            for r in range(CH):
                row_scores = jnp.where(lane == r, _bfly(accs[r], jnp.add),
                                       row_scores)
            live16 = live_v[off16]
            s16 = row_scores * live16
            sc_v[off16] = s16
            m_old = m_run[0]
            m_new = jnp.maximum(m_old, _bfly(s16, jnp.maximum))
            m_run[0] = m_new
            f = jnp.exp(m_old - m_new)
            wvec = jnp.exp(s16 - m_new) * gauss_v[off16] * live16
            wbs = [_bcast(wvec, r) for r in range(CH)]

            def cstep(q, carry):
                off = pl.ds(q * 16, 16)
                acc = ctx_v[off] * f
                for r in range(CH):
                    acc = acc + buf[r, off] * wbs[r]
                ctx_v[off] = acc
                return carry
            lax.fori_loop(0, NQ, cstep, 0)

        gat = [None] * NBUF
        for ci in range(NCH + LOOKAHEAD):
            if ci < NCH:
                k = ci % NBUF
                gat[k] = pltpu.async_copy(
                    table_hbm.at[idx_v.at[pl.ds(ci * CH, CH)]],
                    bufs[k], gsem[k])
            cj = ci - LOOKAHEAD
            if 0 <= cj < NCH:
                kj = cj % NBUF
                gat[kj].wait()
                attend_chunk(bufs[kj], cj)

        mvec = m_run[0]
        dacc = zero16
        for cc in range(WP // 16):
            off = pl.ds(cc * 16, 16)
            win = (lax.iota(jnp.int32, 16) + (cc * 16)) < W
            e = jnp.where(win, jnp.exp(sc_v[off] - mvec), zero16)
            sc_v[off] = e
            dacc = dacc + e
        dinv = jnp.full((16,), 1.0) / _bfly(dacc, jnp.add)

        for cc in range(WP // 16):
            off = pl.ds(cc * 16, 16)
            so_v[off] = sc_v[off] * dinv * gauss_v[off]
        pltpu.sync_copy(so_v, scaled_hbm.at[pl.ds(wid * WP, WP)])

        def nstep(q, carry):
            off = pl.ds(q * 16, 16)
            ctx_v[off] = ctx_v[off] * dinv
            return carry
        lax.fori_loop(0, NQ, nstep, 0)
        pltpu.sync_copy(ctx_v, ctx_hbm.at[pl.ds(wid * H_DIM, H_DIM)])

    return attend_k


def kernel(t, hidden, encoder_outputs, Wp_w, Wp_b, vp_w, vp_b):
    S, B, H = encoder_outputs.shape
    idx2, gauss2, live2 = pl.pallas_call(
        _predict_kernel,
        out_shape=(
            jax.ShapeDtypeStruct((B, WP), jnp.int32),
            jax.ShapeDtypeStruct((B, WP), jnp.float32),
            jax.ShapeDtypeStruct((B, WP), jnp.float32),
        ),
        in_specs=[
            pl.BlockSpec((B, H), lambda: (0, 0)),
            pl.BlockSpec((H, H), lambda: (0, 0)),
            pl.BlockSpec((1, H), lambda: (0, 0)),
            pl.BlockSpec((1, H), lambda: (0, 0)),
            pl.BlockSpec(memory_space=pltpu.SMEM),
        ],
        out_specs=(
            pl.BlockSpec((B, WP), lambda: (0, 0)),
            pl.BlockSpec((B, WP), lambda: (0, 0)),
            pl.BlockSpec((B, WP), lambda: (0, 0)),
        ),
    )(hidden, Wp_w, Wp_b.reshape(1, H), vp_w, vp_b.reshape(1, 1))

    table = encoder_outputs.reshape(S * B, H)
    scaled_flat, ctx_flat = _make_sc_attend()(
        idx2.reshape(B * WP), hidden.reshape(B * H),
        gauss2.reshape(B * WP), live2.reshape(B * WP), table)
    return scaled_flat.reshape(B, WP)[:, :W], ctx_flat.reshape(B, H)


# speedup vs baseline: 1.2086x; 1.2086x over previous
import functools

import jax
import jax.numpy as jnp
from jax import lax
from jax.experimental import pallas as pl
from jax.experimental.pallas import tpu as pltpu
from jax.experimental.pallas import tpu_sc as plsc

D = 128
W = 2 * D + 1
WP = 264
S_DIM, B_DIM, H_DIM = 2048, 32, 1024
CH = 24
NCH = WP // CH
NBUF = 4
LOOKAHEAD = 3


def _predict_kernel(hid_ref, wp_ref, wpb_ref, vp_ref, vpb_ref,
                    p_ref, c_ref, idx_ref):
    h = hid_ref[...]
    wph = lax.dot_general(h, wp_ref[...], (((1,), (1,)), ((), ())),
                          preferred_element_type=jnp.float32)
    wph = jnp.tanh(wph + wpb_ref[...])
    vp8 = jnp.broadcast_to(vp_ref[...], (8, H_DIM))
    z = lax.dot_general(wph, vp8, (((1,), (1,)), ((), ())),
                        preferred_element_type=jnp.float32)[:, :1]
    p = S_DIM * jax.nn.sigmoid(z + vpb_ref[0, 0])
    c = lax.round(p, lax.RoundingMethod.TO_NEAREST_EVEN).astype(jnp.int32)
    p_ref[...] = p
    c_ref[...] = c
    j = lax.broadcasted_iota(jnp.int32, (B_DIM, WP), 1)
    b = lax.broadcasted_iota(jnp.int32, (B_DIM, WP), 0)
    s_abs = jnp.clip(c - D + j, 0, S_DIM - 1)
    idx_ref[...] = s_abs * B_DIM + b


NB_B = 8


def _attn_kernel(p_ref, c_ref, hid_ref, enc_ref, scaled_ref, ctx_ref):
    bpid = pl.program_id(0)
    for i in range(NB_B):
        e_rows = enc_ref[i]
        h = hid_ref[i]
        p = p_ref[bpid * NB_B + i, 0]
        c = c_ref[bpid * NB_B + i, 0]
        j = lax.broadcasted_iota(jnp.int32, (1, WP), 1)
        s_abs = c - D + j
        in_range = (s_abs >= 0) & (s_abs < S_DIM)
        in_win = j < W
        live = in_range & in_win
        scores = lax.dot_general(h, e_rows, (((1,), (1,)), ((), ())),
                                 preferred_element_type=jnp.float32)
        sc = jnp.where(live, scores, 0.0)
        m = jnp.max(jnp.where(in_win, sc, -jnp.inf))
        e = jnp.where(in_win, jnp.exp(sc - m), 0.0)
        attn = e / jnp.sum(e)
        wi = s_abs.astype(jnp.float32)
        gauss = jnp.exp((wi - p) ** 2 * (-1.0 / 8192.0))
        scaled = attn * gauss
        scaled_ref[i] = scaled
        masked = jnp.where(live, scaled, 0.0)
        ctx_ref[i] = lax.dot_general(masked, e_rows, (((1,), (0,)), ((), ())),
                                     preferred_element_type=jnp.float32)


def _make_sc_gather():
    mesh = plsc.VectorSubcoreMesh(core_axis_name="c", subcore_axis_name="s")
    info = plsc.get_sparse_core_info()
    nc = info.num_cores

    @functools.partial(
        pl.kernel, mesh=mesh,
        out_type=jax.ShapeDtypeStruct((B_DIM * WP, H_DIM), jnp.float32),
        scratch_types=(
            [pltpu.VMEM((WP,), jnp.int32)]
            + [pltpu.VMEM((CH, H_DIM), jnp.float32)] * NBUF
            + [pltpu.SemaphoreType.DMA] * (2 * NBUF)
        ),
    )
    def gather_k(idx_hbm, table_hbm, out_hbm, idx_v, *scr):
        bufs = list(scr[:NBUF])
        gsem = list(scr[NBUF:2 * NBUF])
        osem = list(scr[2 * NBUF:])
        wid = lax.axis_index("s") * nc + lax.axis_index("c")
        base = wid * WP
        pltpu.sync_copy(idx_hbm.at[pl.ds(base, WP)], idx_v)
        gat_h = [None] * NBUF
        out_h = [None] * NBUF
        for ci in range(NCH + LOOKAHEAD):
            if ci < NCH:
                k = ci % NBUF
                if out_h[k] is not None:
                    out_h[k].wait()
                gat_h[k] = pltpu.async_copy(
                    table_hbm.at[idx_v.at[pl.ds(ci * CH, CH)]], bufs[k], gsem[k])
            cj = ci - LOOKAHEAD
            if 0 <= cj < NCH:
                kj = cj % NBUF
                gat_h[kj].wait()
                out_h[kj] = pltpu.async_copy(
                    bufs[kj], out_hbm.at[pl.ds(base + cj * CH, CH)], osem[kj])
        for h in out_h:
            if h is not None:
                h.wait()

    return gather_k


def kernel(t, hidden, encoder_outputs, Wp_w, Wp_b, vp_w, vp_b):
    S, B, H = encoder_outputs.shape
    p2, c2, idx2 = pl.pallas_call(
        _predict_kernel,
        out_shape=(
            jax.ShapeDtypeStruct((B, 1), jnp.float32),
            jax.ShapeDtypeStruct((B, 1), jnp.int32),
            jax.ShapeDtypeStruct((B, WP), jnp.int32),
        ),
        in_specs=[
            pl.BlockSpec((B, H), lambda: (0, 0)),
            pl.BlockSpec((H, H), lambda: (0, 0)),
            pl.BlockSpec((1, H), lambda: (0, 0)),
            pl.BlockSpec((1, H), lambda: (0, 0)),
            pl.BlockSpec(memory_space=pltpu.SMEM),
        ],
        out_specs=(
            pl.BlockSpec((B, 1), lambda: (0, 0)),
            pl.BlockSpec((B, 1), lambda: (0, 0)),
            pl.BlockSpec((B, WP), lambda: (0, 0)),
        ),
    )(hidden, Wp_w, Wp_b.reshape(1, H), vp_w, vp_b.reshape(1, 1))

    table = encoder_outputs.reshape(S * B, H)
    enc_flat = _make_sc_gather()(idx2.reshape(B * WP), table)
    enc_local = enc_flat.reshape(B, WP, H)

    scaled_pad, ctx = pl.pallas_call(
        _attn_kernel,
        grid=(B // NB_B,),
        out_shape=(
            jax.ShapeDtypeStruct((B, 1, WP), jnp.float32),
            jax.ShapeDtypeStruct((B, 1, H), jnp.float32),
        ),
        in_specs=[
            pl.BlockSpec(memory_space=pltpu.SMEM),
            pl.BlockSpec(memory_space=pltpu.SMEM),
            pl.BlockSpec((NB_B, 1, H), lambda b: (b, 0, 0)),
            pl.BlockSpec((NB_B, WP, H), lambda b: (b, 0, 0)),
        ],
        out_specs=(
            pl.BlockSpec((NB_B, 1, WP), lambda b: (b, 0, 0)),
            pl.BlockSpec((NB_B, 1, H), lambda b: (b, 0, 0)),
        ),
        compiler_params=pltpu.CompilerParams(
            dimension_semantics=("arbitrary",)),
    )(p2, c2, hidden.reshape(B, 1, H), enc_local)

    return scaled_pad.reshape(B, WP)[:, :W], ctx.reshape(B, H)


# speedup vs baseline: 1.2255x; 1.0140x over previous
import functools

import jax
import jax.numpy as jnp
from jax import lax
from jax.experimental import pallas as pl
from jax.experimental.pallas import tpu as pltpu
from jax.experimental.pallas import tpu_sc as plsc

D = 128
W = 2 * D + 1
WP = 264
S_DIM, B_DIM, H_DIM = 2048, 32, 1024
CH = 24
NCH = WP // CH
NBUF = 4
LOOKAHEAD = 3


def _predict_kernel(hid_ref, wp_ref, wpb_ref, vp_ref, vpb_ref,
                    p_ref, c_ref, idx_ref):
    h = hid_ref[...]
    wph = lax.dot_general(h, wp_ref[...], (((1,), (1,)), ((), ())),
                          preferred_element_type=jnp.float32)
    wph = jnp.tanh(wph + wpb_ref[...])
    vp8 = jnp.broadcast_to(vp_ref[...], (8, H_DIM))
    z = lax.dot_general(wph, vp8, (((1,), (1,)), ((), ())),
                        preferred_element_type=jnp.float32)[:, :1]
    p = S_DIM * jax.nn.sigmoid(z + vpb_ref[0, 0])
    c = lax.round(p, lax.RoundingMethod.TO_NEAREST_EVEN).astype(jnp.int32)
    p_ref[...] = p
    c_ref[...] = c
    j = lax.broadcasted_iota(jnp.int32, (B_DIM, WP), 1)
    b = lax.broadcasted_iota(jnp.int32, (B_DIM, WP), 0)
    s_abs = jnp.clip(c - D + j, 0, S_DIM - 1)
    idx_ref[...] = s_abs * B_DIM + b


NB_B = 4


def _attn_kernel(p_ref, c_ref, hid_ref, enc_ref, scaled_ref, ctx_ref):
    bpid = pl.program_id(0)
    for i in range(NB_B):
        e_rows = enc_ref[i]
        h = hid_ref[i]
        p = p_ref[bpid * NB_B + i, 0]
        c = c_ref[bpid * NB_B + i, 0]
        j = lax.broadcasted_iota(jnp.int32, (1, WP), 1)
        s_abs = c - D + j
        in_range = (s_abs >= 0) & (s_abs < S_DIM)
        in_win = j < W
        live = in_range & in_win
        scores = lax.dot_general(h, e_rows, (((1,), (1,)), ((), ())),
                                 preferred_element_type=jnp.float32)
        sc = jnp.where(live, scores, 0.0)
        m = jnp.max(jnp.where(in_win, sc, -jnp.inf))
        e = jnp.where(in_win, jnp.exp(sc - m), 0.0)
        attn = e / jnp.sum(e)
        wi = s_abs.astype(jnp.float32)
        gauss = jnp.exp((wi - p) ** 2 * (-1.0 / 8192.0))
        scaled = attn * gauss
        scaled_ref[i] = scaled
        masked = jnp.where(live, scaled, 0.0)
        ctx_ref[i] = lax.dot_general(masked, e_rows, (((1,), (0,)), ((), ())),
                                     preferred_element_type=jnp.float32)


def _make_sc_gather():
    mesh = plsc.VectorSubcoreMesh(core_axis_name="c", subcore_axis_name="s")
    info = plsc.get_sparse_core_info()
    nc = info.num_cores

    @functools.partial(
        pl.kernel, mesh=mesh,
        out_type=jax.ShapeDtypeStruct((B_DIM * WP, H_DIM), jnp.float32),
        scratch_types=(
            [pltpu.VMEM((WP,), jnp.int32)]
            + [pltpu.VMEM((CH, H_DIM), jnp.float32)] * NBUF
            + [pltpu.SemaphoreType.DMA] * (2 * NBUF)
        ),
    )
    def gather_k(idx_hbm, table_hbm, out_hbm, idx_v, *scr):
        bufs = list(scr[:NBUF])
        gsem = list(scr[NBUF:2 * NBUF])
        osem = list(scr[2 * NBUF:])
        wid = lax.axis_index("s") * nc + lax.axis_index("c")
        base = wid * WP
        pltpu.sync_copy(idx_hbm.at[pl.ds(base, WP)], idx_v)
        gat_h = [None] * NBUF
        out_h = [None] * NBUF
        for ci in range(NCH + LOOKAHEAD):
            if ci < NCH:
                k = ci % NBUF
                if out_h[k] is not None:
                    out_h[k].wait()
                gat_h[k] = pltpu.async_copy(
                    table_hbm.at[idx_v.at[pl.ds(ci * CH, CH)]], bufs[k], gsem[k])
            cj = ci - LOOKAHEAD
            if 0 <= cj < NCH:
                kj = cj % NBUF
                gat_h[kj].wait()
                out_h[kj] = pltpu.async_copy(
                    bufs[kj], out_hbm.at[pl.ds(base + cj * CH, CH)], osem[kj])
        for h in out_h:
            if h is not None:
                h.wait()

    return gather_k


def kernel(t, hidden, encoder_outputs, Wp_w, Wp_b, vp_w, vp_b):
    S, B, H = encoder_outputs.shape
    p2, c2, idx2 = pl.pallas_call(
        _predict_kernel,
        out_shape=(
            jax.ShapeDtypeStruct((B, 1), jnp.float32),
            jax.ShapeDtypeStruct((B, 1), jnp.int32),
            jax.ShapeDtypeStruct((B, WP), jnp.int32),
        ),
        in_specs=[
            pl.BlockSpec((B, H), lambda: (0, 0)),
            pl.BlockSpec((H, H), lambda: (0, 0)),
            pl.BlockSpec((1, H), lambda: (0, 0)),
            pl.BlockSpec((1, H), lambda: (0, 0)),
            pl.BlockSpec(memory_space=pltpu.SMEM),
        ],
        out_specs=(
            pl.BlockSpec((B, 1), lambda: (0, 0)),
            pl.BlockSpec((B, 1), lambda: (0, 0)),
            pl.BlockSpec((B, WP), lambda: (0, 0)),
        ),
    )(hidden, Wp_w, Wp_b.reshape(1, H), vp_w, vp_b.reshape(1, 1))

    table = encoder_outputs.reshape(S * B, H)
    enc_flat = _make_sc_gather()(idx2.reshape(B * WP), table)
    enc_local = enc_flat.reshape(B, WP, H)

    scaled_pad, ctx = pl.pallas_call(
        _attn_kernel,
        grid=(B // NB_B,),
        out_shape=(
            jax.ShapeDtypeStruct((B, 1, WP), jnp.float32),
            jax.ShapeDtypeStruct((B, 1, H), jnp.float32),
        ),
        in_specs=[
            pl.BlockSpec(memory_space=pltpu.SMEM),
            pl.BlockSpec(memory_space=pltpu.SMEM),
            pl.BlockSpec((NB_B, 1, H), lambda b: (b, 0, 0)),
            pl.BlockSpec((NB_B, WP, H), lambda b: (b, 0, 0)),
        ],
        out_specs=(
            pl.BlockSpec((NB_B, 1, WP), lambda b: (b, 0, 0)),
            pl.BlockSpec((NB_B, 1, H), lambda b: (b, 0, 0)),
        ),
        compiler_params=pltpu.CompilerParams(
            dimension_semantics=("arbitrary",)),
    )(p2, c2, hidden.reshape(B, 1, H), enc_local)

    return scaled_pad.reshape(B, WP)[:, :W], ctx.reshape(B, H)
